# baseline (speedup 1.0000x reference)
import functools

import jax
import jax.numpy as jnp
from jax import lax
from jax.experimental import pallas as pl
from jax.experimental.pallas import tpu as pltpu
from jax.experimental.pallas import tpu_sc as plsc

N = 10000
E = 160000
F_IN = 500
H = 128
C = 3

NC = 2
NS = 16
NW = NC * NS
B = 128

Np = 10240
EC = E // B
CHW = 40
LC = EC - CHW * (NW - 1)
RT = Np // NS

RB = 2048
GRID = Np // RB

_f32 = jnp.float32


def _make_agg(D):
    mesh = plsc.VectorSubcoreMesh(
        core_axis_name="c", subcore_axis_name="s",
        num_cores=NC, num_subcores=NS)

    @functools.partial(
        pl.kernel,
        out_type=jax.ShapeDtypeStruct((NC, Np, D), _f32),
        mesh=mesh,
        scratch_types=[
            pltpu.VMEM((CHW, B), jnp.int32),
            pltpu.VMEM((CHW, B), jnp.int32),
            pltpu.VMEM((2, B, D), _f32),
            pltpu.VMEM_SHARED((Np, D), _f32),
            pltpu.SemaphoreType.DMA((2,)),
            pltpu.SemaphoreType.DMA((3,)),
        ],
    )
    def agg(hs_hbm, src_hbm, dst_hbm, zero_hbm, out_hbm,
            sidx, didx, rows, acc, sem, psem):
        cid = lax.axis_index("c")
        sid = lax.axis_index("s")
        wid = cid * NS + sid
        r0 = pl.multiple_of(sid * RT, 8)
        last = wid == NW - 1
        nch = jnp.where(last, LC, CHW)
        c0 = pl.multiple_of(wid * CHW, 8)
        pltpu.async_copy(zero_hbm.at[pl.ds(r0, RT)], acc.at[pl.ds(r0, RT)],
                         psem.at[2])

        @pl.when(jnp.logical_not(last))
        def _():
            pltpu.async_copy(src_hbm.at[pl.ds(c0, CHW)], sidx, psem.at[0])
            pltpu.async_copy(dst_hbm.at[pl.ds(c0, CHW)], didx, psem.at[1])
            pltpu.make_async_copy(src_hbm.at[pl.ds(c0, CHW)], sidx,
                                  psem.at[0]).wait()

        @pl.when(last)
        def _():
            pltpu.async_copy(src_hbm.at[pl.ds(c0, 8)],
                             sidx.at[pl.ds(0, 8)], psem.at[0])
            pltpu.async_copy(src_hbm.at[pl.ds(c0 + 8, LC - 8)],
                             sidx.at[pl.ds(8, LC - 8)], psem.at[0])
            pltpu.async_copy(dst_hbm.at[pl.ds(c0, 8)],
                             didx.at[pl.ds(0, 8)], psem.at[1])
            pltpu.async_copy(dst_hbm.at[pl.ds(c0 + 8, LC - 8)],
                             didx.at[pl.ds(8, LC - 8)], psem.at[1])
            pltpu.make_async_copy(src_hbm.at[pl.ds(c0, 8)],
                                  sidx.at[pl.ds(0, 8)], psem.at[0]).wait()
            pltpu.make_async_copy(src_hbm.at[pl.ds(c0 + 8, LC - 8)],
                                  sidx.at[pl.ds(8, LC - 8)], psem.at[0]).wait()

        pltpu.async_copy(hs_hbm.at[sidx.at[0]], rows.at[0], sem.at[0])

        @pl.when(jnp.logical_not(last))
        def _():
            pltpu.make_async_copy(dst_hbm.at[pl.ds(c0, CHW)], didx,
                                  psem.at[1]).wait()

        @pl.when(last)
        def _():
            pltpu.make_async_copy(dst_hbm.at[pl.ds(c0, 8)],
                                  didx.at[pl.ds(0, 8)], psem.at[1]).wait()
            pltpu.make_async_copy(dst_hbm.at[pl.ds(c0 + 8, LC - 8)],
                                  didx.at[pl.ds(8, LC - 8)], psem.at[1]).wait()

        pltpu.make_async_copy(zero_hbm.at[pl.ds(r0, RT)],
                              acc.at[pl.ds(r0, RT)], psem.at[2]).wait()
        plsc.subcore_barrier()

        def body(ci, carry):
            par = lax.rem(ci, 2)
            nxt = lax.rem(ci + 1, 2)

            @pl.when(ci + 1 < nch)
            def _():
                pltpu.async_copy(hs_hbm.at[sidx.at[ci + 1]],
                                 rows.at[nxt], sem.at[nxt])

            pltpu.make_async_copy(hs_hbm.at[sidx.at[ci]],
                                  rows.at[par], sem.at[par]).wait()
            pltpu.sync_copy(rows.at[par], acc.at[didx.at[ci]], add=True)
            return carry

        lax.fori_loop(0, nch, body, 0)
        plsc.subcore_barrier()
        pltpu.sync_copy(acc.at[pl.ds(r0, RT)],
                        out_hbm.at[cid, pl.ds(r0, RT)])

    return agg


def _make_deg():
    mesh = plsc.VectorSubcoreMesh(
        core_axis_name="c", subcore_axis_name="s",
        num_cores=NC, num_subcores=NS)

    @functools.partial(
        pl.kernel,
        out_type=jax.ShapeDtypeStruct((NW, Np), _f32),
        mesh=mesh,
        compiler_params=pltpu.CompilerParams(needs_layout_passes=False),
        scratch_types=[
            pltpu.VMEM((CHW, B), jnp.int32),
            pltpu.VMEM((Np,), _f32),
        ],
    )
    def deg(dst_hbm, out_hbm, didx, hist):
        cid = lax.axis_index("c")
        sid = lax.axis_index("s")
        wid = cid * NS + sid
        last = wid == NW - 1
        nch = jnp.where(last, LC, CHW)
        c0 = pl.multiple_of(wid * CHW, 8)

        @pl.when(jnp.logical_not(last))
        def _():
            pltpu.sync_copy(dst_hbm.at[pl.ds(c0, CHW)], didx)

        @pl.when(last)
        def _():
            pltpu.sync_copy(dst_hbm.at[pl.ds(c0, 8)], didx.at[pl.ds(0, 8)])
            pltpu.sync_copy(dst_hbm.at[pl.ds(c0 + 8, LC - 8)],
                            didx.at[pl.ds(8, LC - 8)])

        zero16 = jnp.zeros((16,), _f32)

        def zbody(i, carry):
            hist[pl.ds(i * 16, 16)] = zero16
            return carry

        lax.fori_loop(0, Np // 16, zbody, 0)

        one16 = jnp.ones((16,), _f32)

        def body(ci, carry):
            for j in range(B // 16):
                idx = didx[ci, pl.ds(j * 16, 16)]
                plsc.addupdate_scatter(hist, [idx], one16)
            return carry

        lax.fori_loop(0, nch, body, 0)
        pltpu.sync_copy(hist, out_hbm.at[wid])

    return deg


_make_agg = functools.lru_cache(None)(_make_agg)
_make_deg = functools.lru_cache(None)(_make_deg)


RB1 = 2000


def _k1a_body(x_ref, w_ref, h_ref):
    h_ref[...] = jnp.dot(x_ref[...], w_ref[...], preferred_element_type=_f32)


_k1a = pl.pallas_call(
    _k1a_body,
    grid=(N // RB1,),
    in_specs=[
        pl.BlockSpec((RB1, F_IN), lambda i: (i, 0)),
        pl.BlockSpec((F_IN, H), lambda i: (0, 0)),
    ],
    out_specs=pl.BlockSpec((RB1, H), lambda i: (i, 0)),
    out_shape=jax.ShapeDtypeStruct((Np, H), _f32),
)


def _k1b_body(h_ref, degp_ref, hs_ref, dinv_ref):
    nrows = h_ref.shape[0]
    deg = jnp.sum(degp_ref[...], axis=0)[:, None] + 1.0
    dinv = lax.rsqrt(deg)
    hs_ref[...] = h_ref[...] * dinv
    dinv_ref[...] = jnp.broadcast_to(dinv, (nrows, 16))


_k1b = pl.pallas_call(
    _k1b_body,
    grid=(GRID,),
    in_specs=[
        pl.BlockSpec((RB, H), lambda i: (i, 0)),
        pl.BlockSpec((NW, RB), lambda i: (0, i)),
    ],
    out_specs=[
        pl.BlockSpec((RB, H), lambda i: (i, 0)),
        pl.BlockSpec((RB, 16), lambda i: (i, 0)),
    ],
    out_shape=[
        jax.ShapeDtypeStruct((Np, H), _f32),
        jax.ShapeDtypeStruct((Np, 16), _f32),
    ],
)


def _kc_body(p_ref, hs_ref, dinv_ref, b_ref, w_ref, hs2_ref):
    dv = dinv_ref[:, 0:1]
    s = p_ref[0, :, :] + p_ref[1, :, :] + hs_ref[...]
    a = jnp.maximum(dv * s + b_ref[...], 0.0)
    hs2_ref[...] = jnp.dot(a, w_ref[...], preferred_element_type=_f32) * dv


_kc128 = pl.pallas_call(
    _kc_body,
    grid=(GRID,),
    in_specs=[
        pl.BlockSpec((2, RB, H), lambda i: (0, i, 0)),
        pl.BlockSpec((RB, H), lambda i: (i, 0)),
        pl.BlockSpec((RB, 16), lambda i: (i, 0)),
        pl.BlockSpec((1, H), lambda i: (0, 0)),
        pl.BlockSpec((H, H), lambda i: (0, 0)),
    ],
    out_specs=pl.BlockSpec((RB, H), lambda i: (i, 0)),
    out_shape=jax.ShapeDtypeStruct((Np, H), _f32),
)


def _kc4_body(p_ref, hs_ref, dinv_ref, b_ref, g_ref):
    dv = dinv_ref[:, 0:1]
    s = p_ref[0, :, :] + p_ref[1, :, :] + hs_ref[...]
    g_ref[...] = jnp.maximum(dv * s + b_ref[...], 0.0) * dv


_kc4 = pl.pallas_call(
    _kc4_body,
    grid=(GRID,),
    in_specs=[
        pl.BlockSpec((2, RB, H), lambda i: (0, i, 0)),
        pl.BlockSpec((RB, H), lambda i: (i, 0)),
        pl.BlockSpec((RB, 16), lambda i: (i, 0)),
        pl.BlockSpec((1, H), lambda i: (0, 0)),
    ],
    out_specs=pl.BlockSpec((RB, H), lambda i: (i, 0)),
    out_shape=jax.ShapeDtypeStruct((Np, H), _f32),
)


def _kf_body(p_ref, g_ref, dinv_ref, b_ref, w_ref, out_ref):
    dv = dinv_ref[:, 0:1]
    m = dv * (p_ref[0, :, :] + p_ref[1, :, :] + g_ref[...])
    out_ref[...] = jnp.dot(m, w_ref[...], preferred_element_type=_f32) + b_ref[...]


_kf = pl.pallas_call(
    _kf_body,
    grid=(GRID,),
    in_specs=[
        pl.BlockSpec((2, RB, H), lambda i: (0, i, 0)),
        pl.BlockSpec((RB, H), lambda i: (i, 0)),
        pl.BlockSpec((RB, 16), lambda i: (i, 0)),
        pl.BlockSpec((1, 16), lambda i: (0, 0)),
        pl.BlockSpec((H, 16), lambda i: (0, 0)),
    ],
    out_specs=pl.BlockSpec((RB, 16), lambda i: (i, 0)),
    out_shape=jax.ShapeDtypeStruct((Np, 16), _f32),
)


def kernel(x, edge_index, W1, b1, W2, b2, W3, b3, W4, b4, W5, b5):
    src = edge_index[0].reshape(EC, B)
    dst = edge_index[1].reshape(EC, B)
    zeros128 = jnp.zeros((Np, H), _f32)
    W5p = jnp.pad(W5, ((0, 0), (0, 16 - C)))
    b1r = b1.reshape(1, H)
    b2r = b2.reshape(1, H)
    b3r = b3.reshape(1, H)
    b4r = b4.reshape(1, H)
    b5r = jnp.pad(b5, (0, 16 - C)).reshape(1, 16)

    _deg = _make_deg()
    _agg128 = _make_agg(H)

    h1 = _k1a(x, W1)
    degp = _deg(dst)
    hs1, dinv = _k1b(h1, degp)
    p1 = _agg128(hs1, src, dst, zeros128)
    hs2 = _kc128(p1, hs1, dinv, b1r, W2)
    p2 = _agg128(hs2, src, dst, zeros128)
    hs3 = _kc128(p2, hs2, dinv, b2r, W3)
    p3 = _agg128(hs3, src, dst, zeros128)
    hs4 = _kc128(p3, hs3, dinv, b3r, W4)
    p4 = _agg128(hs4, src, dst, zeros128)
    g4 = _kc4(p4, hs4, dinv, b4r)
    p5 = _agg128(g4, src, dst, zeros128)
    outp = _kf(p5, g4, dinv, b5r, W5p)
    return outp[:N, :C]
